# Initial kernel scaffold; baseline (speedup 1.0000x reference)
import jax
import jax.numpy as jnp
from jax.experimental import pallas as pl


def kernel(x, rel_pos_emb):
    raise NotImplementedError("write your pallas kernel here")


# speedup vs baseline: 2.5581x; 2.5581x over previous
import jax
import jax.numpy as jnp
from jax.experimental import pallas as pl

_MAX_LEN = 8192
_S_BLK = 256


def _add_body(x_ref, emb_ref, out_ref):
    out_ref[...] = x_ref[...] + emb_ref[...][None, :, :]


def kernel(x, rel_pos_emb):
    batch, seq_len, d_model = x.shape
    n_blocks = seq_len // _S_BLK
    grid = (n_blocks,)
    emb_off = _MAX_LEN // _S_BLK
    return pl.pallas_call(
        _add_body,
        grid=grid,
        in_specs=[
            pl.BlockSpec((batch, _S_BLK, d_model), lambda j: (0, j, 0)),
            pl.BlockSpec((_S_BLK, d_model), lambda j: (emb_off + j, 0)),
        ],
        out_specs=pl.BlockSpec((batch, _S_BLK, d_model), lambda j: (0, j, 0)),
        out_shape=jax.ShapeDtypeStruct((batch, seq_len, d_model), x.dtype),
    )(x, rel_pos_emb)


# speedup vs baseline: 2.5885x; 1.0119x over previous
import jax
import jax.numpy as jnp
from jax.experimental import pallas as pl

_MAX_LEN = 8192
_S_BLK = 512


def _add_body(x_ref, emb_ref, out_ref):
    out_ref[...] = x_ref[...] + emb_ref[...][None, :, :]


def kernel(x, rel_pos_emb):
    batch, seq_len, d_model = x.shape
    n_blocks = seq_len // _S_BLK
    grid = (n_blocks,)
    emb_off = _MAX_LEN // _S_BLK
    return pl.pallas_call(
        _add_body,
        grid=grid,
        in_specs=[
            pl.BlockSpec((batch, _S_BLK, d_model), lambda j: (0, j, 0)),
            pl.BlockSpec((_S_BLK, d_model), lambda j: (emb_off + j, 0)),
        ],
        out_specs=pl.BlockSpec((batch, _S_BLK, d_model), lambda j: (0, j, 0)),
        out_shape=jax.ShapeDtypeStruct((batch, seq_len, d_model), x.dtype),
    )(x, rel_pos_emb)
